# baseline (speedup 1.0000x reference)
import jax
import jax.numpy as jnp
from jax.experimental import pallas as pl
from jax.experimental.pallas import tpu as pltpu

_NUM_HEADS = 12
_HEAD_DIM = 64
_POLY_DIM = 256
_MEM_HID = 512
_HIDDEN = 768
_T = 1024


def _atlas_body(coeffs_ref, x_hbm, wq_ref, w1_ref, b1_ref, w2_ref, b2_ref,
                o_hbm, x_buf, o_buf, sem_in, sem_out):
    nch = x_hbm.shape[0] // _T

    def in_cp(k, slot):
        return pltpu.make_async_copy(
            x_hbm.at[pl.ds(k * _T, _T)], x_buf.at[slot], sem_in.at[slot])

    def out_cp(k, slot):
        return pltpu.make_async_copy(
            o_buf.at[slot], o_hbm.at[pl.ds(k * _T, _T)], sem_out.at[slot])

    in_cp(0, 0).start()
    in_cp(1, 1).start()

    def compute(slot, k, i):
        in_cp(k, slot).wait()
        x = x_buf[slot]
        q = jnp.dot(x, wq_ref[...], preferred_element_type=jnp.float32)

        @pl.when(k + 2 < nch)
        def _():
            in_cp(k + 2, slot).start()

        f1 = jnp.clip(q, -10.0, 10.0)
        f2 = f1 * f1
        f3 = f2 * f1

        c0 = coeffs_ref[0]
        w1c = w1_ref[...]
        w1p = w1c[_HEAD_DIM:, :]
        b1 = b1_ref[...] + c0 * jnp.sum(w1c[:_HEAD_DIM, :], axis=0,
                                        keepdims=True)
        w2 = w2_ref[...]
        b2 = b2_ref[...]

        outs = []
        for j in range(_NUM_HEADS):
            sl = slice(j * _HEAD_DIM, (j + 1) * _HEAD_DIM)
            feats = jnp.concatenate([f1[:, sl], f2[:, sl], f3[:, sl]],
                                    axis=-1)
            h = jnp.dot(feats, w1p, preferred_element_type=jnp.float32) + b1
            h = jnp.maximum(h, 0.0)
            outs.append(
                jnp.dot(h, w2, preferred_element_type=jnp.float32) + b2)
        res = jnp.concatenate(outs, axis=-1)

        @pl.when(i > 0)
        def _():
            out_cp(k - 2, slot).wait()

        o_buf[slot] = res
        out_cp(k, slot).start()

    def body(i, carry):
        compute(0, 2 * i, i)
        compute(1, 2 * i + 1, i)
        return carry

    jax.lax.fori_loop(0, nch // 2, body, 0)
    out_cp(nch - 2, 0).wait()
    out_cp(nch - 1, 1).wait()


def kernel(hidden_states, Wq, coeffs, W1, b1, W2, b2):
    B, S, H = hidden_states.shape
    x = hidden_states.reshape(B * S, H)
    rowscale = jnp.repeat(coeffs, _HEAD_DIM)[:, None]
    w1c = W1 * jnp.where(jnp.arange(_POLY_DIM)[:, None] < _HEAD_DIM,
                         1.0, rowscale)
    w2s = W2[:, :_HEAD_DIM]
    b1r = b1.reshape(1, _MEM_HID)
    b2r = b2[:_HEAD_DIM].reshape(1, _HEAD_DIM)

    out = pl.pallas_call(
        _atlas_body,
        in_specs=[
            pl.BlockSpec(memory_space=pltpu.SMEM),
            pl.BlockSpec(memory_space=pl.ANY),
            pl.BlockSpec(memory_space=pltpu.VMEM),
            pl.BlockSpec(memory_space=pltpu.VMEM),
            pl.BlockSpec(memory_space=pltpu.VMEM),
            pl.BlockSpec(memory_space=pltpu.VMEM),
            pl.BlockSpec(memory_space=pltpu.VMEM),
        ],
        out_specs=pl.BlockSpec(memory_space=pl.ANY),
        out_shape=jax.ShapeDtypeStruct((B * S, H), jnp.float32),
        scratch_shapes=[
            pltpu.VMEM((2, _T, _HIDDEN), jnp.float32),
            pltpu.VMEM((2, _T, _HIDDEN), jnp.float32),
            pltpu.SemaphoreType.DMA((2,)),
            pltpu.SemaphoreType.DMA((2,)),
        ],
        name="atlas_attention_fused",
    )(coeffs, x, Wq, w1c, b1r, w2s, b2r)
    return out.reshape(B, S, _NUM_HEADS * _HEAD_DIM)
